# baseline (speedup 1.0000x reference)
import functools

import jax
import jax.numpy as jnp
from jax import lax
from jax.experimental import pallas as pl
from jax.experimental.pallas import tpu as pltpu
from jax.experimental.pallas import tpu_sc as plsc

N = 10000
E = 320000
D = 128
R = 8
B = 4

NC = 2
NS = 16
NW = NC * NS
EPW = E // NW
CHUNK = 80
NCHUNK = EPW // CHUNK
ROWS_PER_TILE = 624
NSTG = 25

BN = 1000
EBN = 80


def _xall_body(wc_ref, w_ref, x_ref, es_ref, et_ref, out_ref, gidx_ref):
  gidx_ref[...] = es_ref[...] * R + et_ref[...]
  xb = x_ref[...].astype(jnp.bfloat16)
  for r in range(R):
    wr = wc_ref[r, 0] * w_ref[0]
    for b in range(1, B):
      wr = wr + wc_ref[r, b] * w_ref[b]
    out_ref[:, r * D:(r + 1) * D] = jnp.dot(
        xb, wr.astype(jnp.bfloat16), preferred_element_type=jnp.float32)


def _combine_body(ws_ref, x_ref, p_ref, out_ref):
  out_ref[...] = (
      jnp.dot(x_ref[...], ws_ref[...], preferred_element_type=jnp.float32)
      + p_ref[0] + p_ref[1])


def _sc_body(xall, gidxh, dsth, normh, out,
             gidx_v, norm_v, didx0, didx1, didx2, rows0, rows1, rows2,
             gsem0, gsem1, gsem2, ssem0, ssem1, ssem2,
             dsem0, dsem1, dsem2, acc):
  cid = lax.axis_index("c")
  sid = lax.axis_index("s")
  wid = sid * NC + cid
  base = wid * EPW
  rows = (rows0, rows1, rows2)
  didx = (didx0, didx1, didx2)
  gsem = (gsem0, gsem1, gsem2)
  ssem = (ssem0, ssem1, ssem2)
  dsem = (dsem0, dsem1, dsem2)

  pltpu.sync_copy(gidxh.at[pl.ds(base, EPW)], gidx_v)

  def zero_body(i, _):
    for k in range(8):
      rows0[i, pl.ds(k * 16, 16)] = jnp.zeros((16,), jnp.float32)
    return 0
  lax.fori_loop(0, CHUNK, zero_body, 0)
  rbase = pl.multiple_of(sid * ROWS_PER_TILE, 16)
  tail_base = NS * ROWS_PER_TILE
  tail = N - tail_base
  nfull = ROWS_PER_TILE // CHUNK
  rem = ROWS_PER_TILE - nfull * CHUNK
  for k in range(nfull):
    pltpu.sync_copy(rows0, acc.at[pl.ds(pl.multiple_of(rbase + k * CHUNK, 8),
                                        CHUNK)])
  pltpu.sync_copy(rows0.at[pl.ds(0, rem)],
                  acc.at[pl.ds(pl.multiple_of(rbase + nfull * CHUNK, 8), rem)])

  @pl.when(sid == NS - 1)
  def _():
    pltpu.sync_copy(rows0.at[pl.ds(0, tail)], acc.at[pl.ds(tail_base, tail)])
  plsc.subcore_barrier()

  def start_didx(c, b):
    pltpu.async_copy(dsth.at[pl.ds(base + c * CHUNK, CHUNK)], didx[b], dsem[b])

  def wait_didx(b):
    pltpu.make_async_copy(dsth.at[pl.ds(base, CHUNK)], didx[b], dsem[b]).wait()

  def start_gather(c, b):
    pltpu.async_copy(xall.at[gidx_v.at[pl.ds(c * CHUNK, CHUNK)]],
                     rows[b], gsem[b])

  def wait_gather(b):
    pltpu.make_async_copy(xall.at[gidx_v.at[pl.ds(0, CHUNK)]],
                          rows[b], gsem[b]).wait()

  def start_scatter(b):
    pltpu.async_copy(rows[b], acc.at[didx[b]], ssem[b], add=True)

  def wait_scatter(b):
    pltpu.make_async_copy(rows[b], acc.at[didx[b]], ssem[b]).wait()

  def chunk_step(c, par):
    nxt = (par + 2) % 3

    @pl.when(lax.rem(c, NSTG) == 0)
    def _():
      pltpu.sync_copy(
          normh.at[pl.ds(base + lax.div(c, NSTG) * (NSTG * CHUNK),
                         NSTG * CHUNK)],
          norm_v)

    @pl.when(c + 2 < NCHUNK)
    def _():
      @pl.when(c > 0)
      def _():
        wait_scatter(nxt)
      start_didx(c + 2, nxt)
      start_gather(c + 2, nxt)
    wait_gather(par)
    cbase = pl.multiple_of(lax.rem(c, NSTG) * CHUNK, 16)

    def scale_body(g, _):
      nv = norm_v[pl.ds(cbase + g * 16, 16)]
      for t in range(16):
        e = g * 16 + t
        for k in range(8):
          rows[par][e, pl.ds(k * 16, 16)] = (
              rows[par][e, pl.ds(k * 16, 16)] * nv[t])
      return 0
    lax.fori_loop(0, CHUNK // 16, scale_body, 0)
    wait_didx(par)
    start_scatter(par)

  start_didx(0, 0)
  start_gather(0, 0)
  start_didx(1, 1)
  start_gather(1, 1)

  def trip_body(q, _):
    c0 = q * 3
    chunk_step(c0, 0)
    chunk_step(c0 + 1, 1)
    chunk_step(c0 + 2, 2)
    return 0
  lax.fori_loop(0, NCHUNK // 3, trip_body, 0)
  chunk_step(NCHUNK - 2, 0)
  chunk_step(NCHUNK - 1, 1)
  wait_scatter(2)
  wait_scatter(0)
  wait_scatter(1)
  plsc.subcore_barrier()

  for k in range(nfull):
    off = pl.multiple_of(rbase + k * CHUNK, 8)
    pltpu.sync_copy(acc.at[pl.ds(off, CHUNK)], out.at[cid, pl.ds(off, CHUNK)])
  off = pl.multiple_of(rbase + nfull * CHUNK, 8)
  pltpu.sync_copy(acc.at[pl.ds(off, rem)], out.at[cid, pl.ds(off, rem)])

  @pl.when(sid == NS - 1)
  def _():
    pltpu.sync_copy(acc.at[pl.ds(tail_base, tail)],
                    out.at[cid, pl.ds(tail_base, tail)])


def kernel(x, norm, rel_emb, weight, w_comp, self_loop_weight,
           edge_src, edge_dst, edge_type, edge_label):
  del rel_emb, edge_label

  grid = N // BN
  es2 = edge_src.reshape(E // 400, 400)
  et2 = edge_type.reshape(E // 400, 400)
  x_all, gidx = pl.pallas_call(
      _xall_body,
      grid=(grid,),
      in_specs=[
          pl.BlockSpec(memory_space=pltpu.SMEM),
          pl.BlockSpec((B, D, D), lambda i: (0, 0, 0)),
          pl.BlockSpec((BN, D), lambda i: (i, 0)),
          pl.BlockSpec((EBN, 400), lambda i: (i, 0)),
          pl.BlockSpec((EBN, 400), lambda i: (i, 0)),
      ],
      out_specs=[
          pl.BlockSpec((BN, R * D), lambda i: (i, 0)),
          pl.BlockSpec((EBN, 400), lambda i: (i, 0)),
      ],
      out_shape=[
          jax.ShapeDtypeStruct((N, R * D), jnp.float32),
          jax.ShapeDtypeStruct((E // 400, 400), jnp.int32),
      ],
      compiler_params=pltpu.CompilerParams(
          dimension_semantics=("arbitrary",)),
  )(w_comp, weight, x, es2, et2)
  xall_flat = x_all.reshape(N * R, D)
  gidx_flat = gidx.reshape(E)

  sc_kernel = functools.partial(
      pl.kernel,
      out_type=jax.ShapeDtypeStruct((NC, N, D), jnp.float32),
      mesh=plsc.VectorSubcoreMesh(core_axis_name="c", subcore_axis_name="s"),
      scratch_types=(
          [pltpu.VMEM((EPW,), jnp.int32)]
          + [pltpu.VMEM((NSTG * CHUNK,), jnp.float32)]
          + [pltpu.VMEM((CHUNK,), jnp.int32) for _ in range(3)]
          + [pltpu.VMEM((CHUNK, D), jnp.float32) for _ in range(3)]
          + [pltpu.SemaphoreType.DMA for _ in range(9)]
          + [pltpu.VMEM_SHARED((N, D), jnp.float32)]
      ),
  )(_sc_body)
  partials = sc_kernel(xall_flat, gidx_flat, edge_dst, norm)

  h = pl.pallas_call(
      _combine_body,
      grid=(grid,),
      in_specs=[
          pl.BlockSpec((D, D), lambda i: (0, 0)),
          pl.BlockSpec((BN, D), lambda i: (i, 0)),
          pl.BlockSpec((NC, BN, D), lambda i: (0, i, 0)),
      ],
      out_specs=pl.BlockSpec((BN, D), lambda i: (i, 0)),
      out_shape=jax.ShapeDtypeStruct((N, D), jnp.float32),
      compiler_params=pltpu.CompilerParams(
          dimension_semantics=("arbitrary",)),
  )(self_loop_weight, x, partials)

  return h, h[:, None, :]


# speedup vs baseline: 1.2388x; 1.2388x over previous
import functools

import jax
import jax.numpy as jnp
from jax import lax
from jax.experimental import pallas as pl
from jax.experimental.pallas import tpu as pltpu
from jax.experimental.pallas import tpu_sc as plsc

N = 10000
E = 320000
D = 128
R = 8
B = 4

NC = 2
NS = 16
NW = NC * NS
EPW = E // NW
CHUNK = 80
NCHUNK = EPW // CHUNK
ROWS_PER_TILE = 624
NSTG = 25

BN = 1000
EBN = 80


def _xall_body(wc_ref, w_ref, x_ref, es_ref, et_ref, out_ref, gidx_ref):
  gidx_ref[...] = et_ref[...] * N + es_ref[...]
  xb = x_ref[...]
  for r in range(R):
    wr = wc_ref[r, 0] * w_ref[0]
    for b in range(1, B):
      wr = wr + wc_ref[r, b] * w_ref[b]
    out_ref[r] = jnp.dot(xb, wr, preferred_element_type=jnp.float32)


def _combine_body(ws_ref, x_ref, p_ref, out_ref, rep_ref):
  h = (jnp.dot(x_ref[...], ws_ref[...], preferred_element_type=jnp.float32)
       + p_ref[0] + p_ref[1])
  out_ref[...] = h
  rep_ref[:, 0, :] = h


def _sc_body(xall, gidxh, dsth, normh, out,
             gidx_v, norm_v, didx0, didx1, didx2, rows0, rows1, rows2,
             gsem0, gsem1, gsem2, ssem0, ssem1, ssem2,
             dsem0, dsem1, dsem2, acc):
  cid = lax.axis_index("c")
  sid = lax.axis_index("s")
  wid = sid * NC + cid
  base = wid * EPW
  rows = (rows0, rows1, rows2)
  didx = (didx0, didx1, didx2)
  gsem = (gsem0, gsem1, gsem2)
  ssem = (ssem0, ssem1, ssem2)
  dsem = (dsem0, dsem1, dsem2)

  pltpu.sync_copy(gidxh.at[pl.ds(base, EPW)], gidx_v)

  def zero_body(i, _):
    for k in range(8):
      rows0[i, pl.ds(k * 16, 16)] = jnp.zeros((16,), jnp.float32)
    return 0
  lax.fori_loop(0, CHUNK, zero_body, 0)
  rbase = pl.multiple_of(sid * ROWS_PER_TILE, 16)
  tail_base = NS * ROWS_PER_TILE
  tail = N - tail_base
  nfull = ROWS_PER_TILE // CHUNK
  rem = ROWS_PER_TILE - nfull * CHUNK
  for k in range(nfull):
    pltpu.sync_copy(rows0, acc.at[pl.ds(pl.multiple_of(rbase + k * CHUNK, 8),
                                        CHUNK)])
  pltpu.sync_copy(rows0.at[pl.ds(0, rem)],
                  acc.at[pl.ds(pl.multiple_of(rbase + nfull * CHUNK, 8), rem)])

  @pl.when(sid == NS - 1)
  def _():
    pltpu.sync_copy(rows0.at[pl.ds(0, tail)], acc.at[pl.ds(tail_base, tail)])
  plsc.subcore_barrier()

  def start_didx(c, b):
    pltpu.async_copy(dsth.at[pl.ds(base + c * CHUNK, CHUNK)], didx[b], dsem[b])

  def wait_didx(b):
    pltpu.make_async_copy(dsth.at[pl.ds(base, CHUNK)], didx[b], dsem[b]).wait()

  def start_gather(c, b):
    pltpu.async_copy(xall.at[gidx_v.at[pl.ds(c * CHUNK, CHUNK)]],
                     rows[b], gsem[b])

  def wait_gather(b):
    pltpu.make_async_copy(xall.at[gidx_v.at[pl.ds(0, CHUNK)]],
                          rows[b], gsem[b]).wait()

  def start_scatter(b):
    pltpu.async_copy(rows[b], acc.at[didx[b]], ssem[b], add=True)

  def wait_scatter(b):
    pltpu.make_async_copy(rows[b], acc.at[didx[b]], ssem[b]).wait()

  def chunk_step(c, par):
    nxt = (par + 2) % 3

    @pl.when(lax.rem(c, NSTG) == 0)
    def _():
      pltpu.sync_copy(
          normh.at[pl.ds(base + lax.div(c, NSTG) * (NSTG * CHUNK),
                         NSTG * CHUNK)],
          norm_v)

    @pl.when(c + 2 < NCHUNK)
    def _():
      @pl.when(c > 0)
      def _():
        wait_scatter(nxt)
      start_didx(c + 2, nxt)
      start_gather(c + 2, nxt)
    wait_gather(par)
    cbase = pl.multiple_of(lax.rem(c, NSTG) * CHUNK, 16)

    def scale_body(g, _):
      nv = norm_v[pl.ds(cbase + g * 16, 16)]
      for t in range(16):
        e = g * 16 + t
        for k in range(8):
          rows[par][e, pl.ds(k * 16, 16)] = (
              rows[par][e, pl.ds(k * 16, 16)] * nv[t])
      return 0
    lax.fori_loop(0, CHUNK // 16, scale_body, 0)
    wait_didx(par)
    start_scatter(par)

  start_didx(0, 0)
  start_gather(0, 0)
  start_didx(1, 1)
  start_gather(1, 1)

  def trip_body(q, _):
    c0 = q * 3
    chunk_step(c0, 0)
    chunk_step(c0 + 1, 1)
    chunk_step(c0 + 2, 2)
    return 0
  lax.fori_loop(0, NCHUNK // 3, trip_body, 0)
  chunk_step(NCHUNK - 2, 0)
  chunk_step(NCHUNK - 1, 1)
  wait_scatter(2)
  wait_scatter(0)
  wait_scatter(1)
  plsc.subcore_barrier()

  for k in range(nfull):
    off = pl.multiple_of(rbase + k * CHUNK, 8)
    pltpu.sync_copy(acc.at[pl.ds(off, CHUNK)], out.at[cid, pl.ds(off, CHUNK)])
  off = pl.multiple_of(rbase + nfull * CHUNK, 8)
  pltpu.sync_copy(acc.at[pl.ds(off, rem)], out.at[cid, pl.ds(off, rem)])

  @pl.when(sid == NS - 1)
  def _():
    pltpu.sync_copy(acc.at[pl.ds(tail_base, tail)],
                    out.at[cid, pl.ds(tail_base, tail)])


def kernel(x, norm, rel_emb, weight, w_comp, self_loop_weight,
           edge_src, edge_dst, edge_type, edge_label):
  del rel_emb, edge_label

  grid = N // BN
  es2 = edge_src.reshape(E // 400, 400)
  et2 = edge_type.reshape(E // 400, 400)
  x_all, gidx = pl.pallas_call(
      _xall_body,
      grid=(grid,),
      in_specs=[
          pl.BlockSpec(memory_space=pltpu.SMEM),
          pl.BlockSpec((B, D, D), lambda i: (0, 0, 0)),
          pl.BlockSpec((BN, D), lambda i: (i, 0)),
          pl.BlockSpec((EBN, 400), lambda i: (i, 0)),
          pl.BlockSpec((EBN, 400), lambda i: (i, 0)),
      ],
      out_specs=[
          pl.BlockSpec((R, BN, D), lambda i: (0, i, 0)),
          pl.BlockSpec((EBN, 400), lambda i: (i, 0)),
      ],
      out_shape=[
          jax.ShapeDtypeStruct((R, N, D), jnp.float32),
          jax.ShapeDtypeStruct((E // 400, 400), jnp.int32),
      ],
      compiler_params=pltpu.CompilerParams(
          dimension_semantics=("arbitrary",)),
  )(w_comp, weight, x, es2, et2)
  xall_flat = x_all.reshape(R * N, D)
  gidx_flat = gidx.reshape(E)

  sc_kernel = functools.partial(
      pl.kernel,
      out_type=jax.ShapeDtypeStruct((NC, N, D), jnp.float32),
      mesh=plsc.VectorSubcoreMesh(core_axis_name="c", subcore_axis_name="s"),
      scratch_types=(
          [pltpu.VMEM((EPW,), jnp.int32)]
          + [pltpu.VMEM((NSTG * CHUNK,), jnp.float32)]
          + [pltpu.VMEM((CHUNK,), jnp.int32) for _ in range(3)]
          + [pltpu.VMEM((CHUNK, D), jnp.float32) for _ in range(3)]
          + [pltpu.SemaphoreType.DMA for _ in range(9)]
          + [pltpu.VMEM_SHARED((N, D), jnp.float32)]
      ),
  )(_sc_body)
  partials = sc_kernel(xall_flat, gidx_flat, edge_dst, norm)

  h, rep = pl.pallas_call(
      _combine_body,
      grid=(grid,),
      in_specs=[
          pl.BlockSpec((D, D), lambda i: (0, 0)),
          pl.BlockSpec((BN, D), lambda i: (i, 0)),
          pl.BlockSpec((NC, BN, D), lambda i: (0, i, 0)),
      ],
      out_specs=[
          pl.BlockSpec((BN, D), lambda i: (i, 0)),
          pl.BlockSpec((BN, 1, D), lambda i: (i, 0, 0)),
      ],
      out_shape=[
          jax.ShapeDtypeStruct((N, D), jnp.float32),
          jax.ShapeDtypeStruct((N, 1, D), jnp.float32),
      ],
      compiler_params=pltpu.CompilerParams(
          dimension_semantics=("arbitrary",)),
  )(self_loop_weight, x, partials)

  return h, rep


# speedup vs baseline: 1.3097x; 1.0572x over previous
import functools

import jax
import jax.numpy as jnp
from jax import lax
from jax.experimental import pallas as pl
from jax.experimental.pallas import tpu as pltpu
from jax.experimental.pallas import tpu_sc as plsc

N = 10000
E = 320000
D = 128
R = 8
B = 4

NC = 2
NS = 16
NW = NC * NS
EPW = E // NW
CHUNK = 80
NCHUNK = EPW // CHUNK
ROWS_PER_TILE = 624
NSTG = 25

BN = 1000
EBN = 80


def _xall_body(wc_ref, w_ref, x_ref, es_ref, et_ref, out_ref, gidx_ref):
  @pl.when(pl.program_id(0) == 0)
  def _():
    gidx_ref[...] = et_ref[...] * N + es_ref[...]
  xb = x_ref[...]
  for r in range(R):
    wr = wc_ref[r, 0] * w_ref[0]
    for b in range(1, B):
      wr = wr + wc_ref[r, b] * w_ref[b]
    out_ref[r] = jnp.dot(xb, wr, preferred_element_type=jnp.float32)


def _combine_body(ws_ref, x_ref, p_ref, out_ref, rep_ref):
  h = (jnp.dot(x_ref[...], ws_ref[...], preferred_element_type=jnp.float32)
       + p_ref[0] + p_ref[1])
  out_ref[...] = h
  rep_ref[:, 0, :] = h


def _sc_body(xall, gidxh, dsth, normh, out,
             gidx_v, norm_v, didx0, didx1, didx2, rows0, rows1, rows2,
             gsem0, gsem1, gsem2, ssem0, ssem1, ssem2,
             dsem0, dsem1, dsem2, acc):
  cid = lax.axis_index("c")
  sid = lax.axis_index("s")
  wid = sid * NC + cid
  base = wid * EPW
  rows = (rows0, rows1, rows2)
  didx = (didx0, didx1, didx2)
  gsem = (gsem0, gsem1, gsem2)
  ssem = (ssem0, ssem1, ssem2)
  dsem = (dsem0, dsem1, dsem2)

  pltpu.sync_copy(gidxh.at[pl.ds(base, EPW)], gidx_v)

  def zero_body(i, _):
    for k in range(8):
      rows0[i, pl.ds(k * 16, 16)] = jnp.zeros((16,), jnp.float32)
    return 0
  lax.fori_loop(0, CHUNK, zero_body, 0)
  rbase = pl.multiple_of(sid * ROWS_PER_TILE, 16)
  tail_base = NS * ROWS_PER_TILE
  tail = N - tail_base
  nfull = ROWS_PER_TILE // CHUNK
  rem = ROWS_PER_TILE - nfull * CHUNK
  for k in range(nfull):
    pltpu.sync_copy(rows0, acc.at[pl.ds(pl.multiple_of(rbase + k * CHUNK, 8),
                                        CHUNK)])
  pltpu.sync_copy(rows0.at[pl.ds(0, rem)],
                  acc.at[pl.ds(pl.multiple_of(rbase + nfull * CHUNK, 8), rem)])

  @pl.when(sid == NS - 1)
  def _():
    pltpu.sync_copy(rows0.at[pl.ds(0, tail)], acc.at[pl.ds(tail_base, tail)])
  plsc.subcore_barrier()

  def start_didx(c, b):
    pltpu.async_copy(dsth.at[pl.ds(base + c * CHUNK, CHUNK)], didx[b], dsem[b])

  def wait_didx(b):
    pltpu.make_async_copy(dsth.at[pl.ds(base, CHUNK)], didx[b], dsem[b]).wait()

  def start_gather(c, b):
    pltpu.async_copy(xall.at[gidx_v.at[pl.ds(c * CHUNK, CHUNK)]],
                     rows[b], gsem[b])

  def wait_gather(b):
    pltpu.make_async_copy(xall.at[gidx_v.at[pl.ds(0, CHUNK)]],
                          rows[b], gsem[b]).wait()

  def start_scatter(b):
    pltpu.async_copy(rows[b], acc.at[didx[b]], ssem[b], add=True)

  def wait_scatter(b):
    pltpu.make_async_copy(rows[b], acc.at[didx[b]], ssem[b]).wait()

  def chunk_step(c, par):
    nxt = (par + 2) % 3

    @pl.when(lax.rem(c, NSTG) == 0)
    def _():
      pltpu.sync_copy(
          normh.at[pl.ds(base + lax.div(c, NSTG) * (NSTG * CHUNK),
                         NSTG * CHUNK)],
          norm_v)

    @pl.when(c + 2 < NCHUNK)
    def _():
      @pl.when(c > 0)
      def _():
        wait_scatter(nxt)
      start_didx(c + 2, nxt)
      start_gather(c + 2, nxt)
    wait_gather(par)
    cbase = pl.multiple_of(lax.rem(c, NSTG) * CHUNK, 16)

    def scale_body(g, _):
      nv = norm_v[pl.ds(cbase + g * 16, 16)]
      for t in range(16):
        e = g * 16 + t
        for k in range(8):
          rows[par][e, pl.ds(k * 16, 16)] = (
              rows[par][e, pl.ds(k * 16, 16)] * nv[t])
      return 0
    lax.fori_loop(0, CHUNK // 16, scale_body, 0)
    wait_didx(par)
    start_scatter(par)

  start_didx(0, 0)
  start_gather(0, 0)
  start_didx(1, 1)
  start_gather(1, 1)

  def trip_body(q, _):
    c0 = q * 3
    chunk_step(c0, 0)
    chunk_step(c0 + 1, 1)
    chunk_step(c0 + 2, 2)
    return 0
  lax.fori_loop(0, NCHUNK // 3, trip_body, 0)
  chunk_step(NCHUNK - 2, 0)
  chunk_step(NCHUNK - 1, 1)
  wait_scatter(2)
  wait_scatter(0)
  wait_scatter(1)
  plsc.subcore_barrier()

  for k in range(nfull):
    off = pl.multiple_of(rbase + k * CHUNK, 8)
    pltpu.sync_copy(acc.at[pl.ds(off, CHUNK)], out.at[cid, pl.ds(off, CHUNK)])
  off = pl.multiple_of(rbase + nfull * CHUNK, 8)
  pltpu.sync_copy(acc.at[pl.ds(off, rem)], out.at[cid, pl.ds(off, rem)])

  @pl.when(sid == NS - 1)
  def _():
    pltpu.sync_copy(acc.at[pl.ds(tail_base, tail)],
                    out.at[cid, pl.ds(tail_base, tail)])


def kernel(x, norm, rel_emb, weight, w_comp, self_loop_weight,
           edge_src, edge_dst, edge_type, edge_label):
  del rel_emb, edge_label

  grid = N // BN
  x_all, gidx = pl.pallas_call(
      _xall_body,
      grid=(grid,),
      in_specs=[
          pl.BlockSpec(memory_space=pltpu.SMEM),
          pl.BlockSpec((B, D, D), lambda i: (0, 0, 0)),
          pl.BlockSpec((BN, D), lambda i: (i, 0)),
          pl.BlockSpec((E,), lambda i: (0,)),
          pl.BlockSpec((E,), lambda i: (0,)),
      ],
      out_specs=[
          pl.BlockSpec((R, BN, D), lambda i: (0, i, 0)),
          pl.BlockSpec((E,), lambda i: (0,)),
      ],
      out_shape=[
          jax.ShapeDtypeStruct((R, N, D), jnp.float32),
          jax.ShapeDtypeStruct((E,), jnp.int32),
      ],
      compiler_params=pltpu.CompilerParams(
          dimension_semantics=("arbitrary",)),
  )(w_comp, weight, x, edge_src, edge_type)
  xall_flat = x_all.reshape(R * N, D)

  sc_kernel = functools.partial(
      pl.kernel,
      out_type=jax.ShapeDtypeStruct((NC, N, D), jnp.float32),
      mesh=plsc.VectorSubcoreMesh(core_axis_name="c", subcore_axis_name="s"),
      scratch_types=(
          [pltpu.VMEM((EPW,), jnp.int32)]
          + [pltpu.VMEM((NSTG * CHUNK,), jnp.float32)]
          + [pltpu.VMEM((CHUNK,), jnp.int32) for _ in range(3)]
          + [pltpu.VMEM((CHUNK, D), jnp.float32) for _ in range(3)]
          + [pltpu.SemaphoreType.DMA for _ in range(9)]
          + [pltpu.VMEM_SHARED((N, D), jnp.float32)]
      ),
  )(_sc_body)
  partials = sc_kernel(xall_flat, gidx, edge_dst, norm)

  h, rep = pl.pallas_call(
      _combine_body,
      grid=(grid,),
      in_specs=[
          pl.BlockSpec((D, D), lambda i: (0, 0)),
          pl.BlockSpec((BN, D), lambda i: (i, 0)),
          pl.BlockSpec((NC, BN, D), lambda i: (0, i, 0)),
      ],
      out_specs=[
          pl.BlockSpec((BN, D), lambda i: (i, 0)),
          pl.BlockSpec((BN, 1, D), lambda i: (i, 0, 0)),
      ],
      out_shape=[
          jax.ShapeDtypeStruct((N, D), jnp.float32),
          jax.ShapeDtypeStruct((N, 1, D), jnp.float32),
      ],
      compiler_params=pltpu.CompilerParams(
          dimension_semantics=("arbitrary",)),
  )(self_loop_weight, x, partials)

  return h, rep


# speedup vs baseline: 1.3274x; 1.0135x over previous
import functools

import jax
import jax.numpy as jnp
from jax import lax
from jax.experimental import pallas as pl
from jax.experimental.pallas import tpu as pltpu
from jax.experimental.pallas import tpu_sc as plsc

N = 10000
E = 320000
D = 128
R = 8
B = 4

NC = 2
NS = 16
NW = NC * NS
EPW = E // NW
CHUNK = 80
NCHUNK = EPW // CHUNK
ROWS_PER_TILE = 624
NSTG = 25

BN = 1000
EBN = 80


def _xall_body(wc_ref, w_ref, x_ref, es_ref, et_ref, out_ref, gidx_ref):
  @pl.when(pl.program_id(0) == 0)
  def _():
    gidx_ref[...] = et_ref[...] * N + es_ref[...]
  xb = x_ref[...]
  for r in range(R):
    wr = wc_ref[r, 0] * w_ref[0]
    for b in range(1, B):
      wr = wr + wc_ref[r, b] * w_ref[b]
    out_ref[r] = jnp.dot(xb, wr, preferred_element_type=jnp.float32)


def _combine_body(ws_ref, x_ref, p_ref, out_ref, rep_ref):
  h = (jnp.dot(x_ref[...], ws_ref[...], preferred_element_type=jnp.float32)
       + p_ref[0] + p_ref[1])
  out_ref[...] = h
  rep_ref[:, 0, :] = h


def _sc_body(xall, gidxh, dsth, normh, out,
             gidx_v, norm_v, didx0, didx1, didx2, rows0, rows1, rows2,
             gsem0, gsem1, gsem2, ssem0, ssem1, ssem2,
             dsem0, dsem1, dsem2, isem, acc):
  cid = lax.axis_index("c")
  sid = lax.axis_index("s")
  wid = sid * NC + cid
  base = wid * EPW
  rows = (rows0, rows1, rows2)
  didx = (didx0, didx1, didx2)
  gsem = (gsem0, gsem1, gsem2)
  ssem = (ssem0, ssem1, ssem2)
  dsem = (dsem0, dsem1, dsem2)

  def start_didx(c, b):
    pltpu.async_copy(dsth.at[pl.ds(base + c * CHUNK, CHUNK)], didx[b], dsem[b])

  def wait_didx(b):
    pltpu.make_async_copy(dsth.at[pl.ds(base, CHUNK)], didx[b], dsem[b]).wait()

  def start_gather(c, b):
    pltpu.async_copy(xall.at[gidx_v.at[pl.ds(c * CHUNK, CHUNK)]],
                     rows[b], gsem[b])

  def wait_gather(b):
    pltpu.make_async_copy(xall.at[gidx_v.at[pl.ds(0, CHUNK)]],
                          rows[b], gsem[b]).wait()

  def start_scatter(b):
    pltpu.async_copy(rows[b], acc.at[didx[b]], ssem[b], add=True)

  def wait_scatter(b):
    pltpu.make_async_copy(rows[b], acc.at[didx[b]], ssem[b]).wait()

  pltpu.async_copy(gidxh.at[pl.ds(base, EPW)], gidx_v, isem)

  def zero_body(i, _):
    for k in range(8):
      rows0[i, pl.ds(k * 16, 16)] = jnp.zeros((16,), jnp.float32)
    return 0
  lax.fori_loop(0, CHUNK, zero_body, 0)
  pltpu.make_async_copy(gidxh.at[pl.ds(base, EPW)], gidx_v, isem).wait()
  start_didx(0, 0)
  start_gather(0, 0)
  start_didx(1, 1)
  start_gather(1, 1)
  rbase = pl.multiple_of(sid * ROWS_PER_TILE, 16)
  tail_base = NS * ROWS_PER_TILE
  tail = N - tail_base
  nfull = ROWS_PER_TILE // CHUNK
  rem = ROWS_PER_TILE - nfull * CHUNK
  zslices = [(pl.multiple_of(rbase + k * CHUNK, 8), CHUNK)
             for k in range(nfull)]
  zslices.append((pl.multiple_of(rbase + nfull * CHUNK, 8), rem))
  for off, n in zslices:
    pltpu.async_copy(rows0.at[pl.ds(0, n)], acc.at[pl.ds(off, n)], isem)

  @pl.when(sid == NS - 1)
  def _():
    pltpu.async_copy(rows0.at[pl.ds(0, tail)], acc.at[pl.ds(tail_base, tail)],
                     isem)
  for off, n in zslices:
    pltpu.make_async_copy(rows0.at[pl.ds(0, n)], acc.at[pl.ds(off, n)],
                          isem).wait()

  @pl.when(sid == NS - 1)
  def _():
    pltpu.make_async_copy(rows0.at[pl.ds(0, tail)],
                          acc.at[pl.ds(tail_base, tail)], isem).wait()
  plsc.subcore_barrier()

  def chunk_step(c, par):
    nxt = (par + 2) % 3

    @pl.when(lax.rem(c, NSTG) == 0)
    def _():
      pltpu.sync_copy(
          normh.at[pl.ds(base + lax.div(c, NSTG) * (NSTG * CHUNK),
                         NSTG * CHUNK)],
          norm_v)

    @pl.when(c + 2 < NCHUNK)
    def _():
      @pl.when(c > 0)
      def _():
        wait_scatter(nxt)
      start_didx(c + 2, nxt)
      start_gather(c + 2, nxt)
    wait_gather(par)
    cbase = pl.multiple_of(lax.rem(c, NSTG) * CHUNK, 16)

    def scale_body(g, _):
      nv = norm_v[pl.ds(cbase + g * 16, 16)]
      for t in range(16):
        e = g * 16 + t
        for k in range(8):
          rows[par][e, pl.ds(k * 16, 16)] = (
              rows[par][e, pl.ds(k * 16, 16)] * nv[t])
      return 0
    lax.fori_loop(0, CHUNK // 16, scale_body, 0)
    wait_didx(par)
    start_scatter(par)

  def trip_body(q, _):
    c0 = q * 3
    chunk_step(c0, 0)
    chunk_step(c0 + 1, 1)
    chunk_step(c0 + 2, 2)
    return 0
  lax.fori_loop(0, NCHUNK // 3, trip_body, 0)
  chunk_step(NCHUNK - 2, 0)
  chunk_step(NCHUNK - 1, 1)
  wait_scatter(2)
  wait_scatter(0)
  wait_scatter(1)
  plsc.subcore_barrier()

  for off, n in zslices:
    pltpu.async_copy(acc.at[pl.ds(off, n)], out.at[cid, pl.ds(off, n)], isem)

  @pl.when(sid == NS - 1)
  def _():
    pltpu.async_copy(acc.at[pl.ds(tail_base, tail)],
                     out.at[cid, pl.ds(tail_base, tail)], isem)
  for off, n in zslices:
    pltpu.make_async_copy(acc.at[pl.ds(off, n)],
                          out.at[cid, pl.ds(off, n)], isem).wait()

  @pl.when(sid == NS - 1)
  def _():
    pltpu.make_async_copy(acc.at[pl.ds(tail_base, tail)],
                          out.at[cid, pl.ds(tail_base, tail)], isem).wait()


def kernel(x, norm, rel_emb, weight, w_comp, self_loop_weight,
           edge_src, edge_dst, edge_type, edge_label):
  del rel_emb, edge_label

  grid = N // BN
  x_all, gidx = pl.pallas_call(
      _xall_body,
      grid=(grid,),
      in_specs=[
          pl.BlockSpec(memory_space=pltpu.SMEM),
          pl.BlockSpec((B, D, D), lambda i: (0, 0, 0)),
          pl.BlockSpec((BN, D), lambda i: (i, 0)),
          pl.BlockSpec((E,), lambda i: (0,)),
          pl.BlockSpec((E,), lambda i: (0,)),
      ],
      out_specs=[
          pl.BlockSpec((R, BN, D), lambda i: (0, i, 0)),
          pl.BlockSpec((E,), lambda i: (0,)),
      ],
      out_shape=[
          jax.ShapeDtypeStruct((R, N, D), jnp.float32),
          jax.ShapeDtypeStruct((E,), jnp.int32),
      ],
      compiler_params=pltpu.CompilerParams(
          dimension_semantics=("arbitrary",)),
  )(w_comp, weight, x, edge_src, edge_type)
  xall_flat = x_all.reshape(R * N, D)

  sc_kernel = functools.partial(
      pl.kernel,
      out_type=jax.ShapeDtypeStruct((NC, N, D), jnp.float32),
      mesh=plsc.VectorSubcoreMesh(core_axis_name="c", subcore_axis_name="s"),
      scratch_types=(
          [pltpu.VMEM((EPW,), jnp.int32)]
          + [pltpu.VMEM((NSTG * CHUNK,), jnp.float32)]
          + [pltpu.VMEM((CHUNK,), jnp.int32) for _ in range(3)]
          + [pltpu.VMEM((CHUNK, D), jnp.float32) for _ in range(3)]
          + [pltpu.SemaphoreType.DMA for _ in range(10)]
          + [pltpu.VMEM_SHARED((N, D), jnp.float32)]
      ),
  )(_sc_body)
  partials = sc_kernel(xall_flat, gidx, edge_dst, norm)

  h, rep = pl.pallas_call(
      _combine_body,
      grid=(grid,),
      in_specs=[
          pl.BlockSpec((D, D), lambda i: (0, 0)),
          pl.BlockSpec((BN, D), lambda i: (i, 0)),
          pl.BlockSpec((NC, BN, D), lambda i: (0, i, 0)),
      ],
      out_specs=[
          pl.BlockSpec((BN, D), lambda i: (i, 0)),
          pl.BlockSpec((BN, 1, D), lambda i: (i, 0, 0)),
      ],
      out_shape=[
          jax.ShapeDtypeStruct((N, D), jnp.float32),
          jax.ShapeDtypeStruct((N, 1, D), jnp.float32),
      ],
      compiler_params=pltpu.CompilerParams(
          dimension_semantics=("arbitrary",)),
  )(self_loop_weight, x, partials)

  return h, rep


# speedup vs baseline: 1.3289x; 1.0012x over previous
import functools

import jax
import jax.numpy as jnp
from jax import lax
from jax.experimental import pallas as pl
from jax.experimental.pallas import tpu as pltpu
from jax.experimental.pallas import tpu_sc as plsc

N = 10000
E = 320000
D = 128
R = 8
B = 4

NC = 2
NS = 16
NW = NC * NS
EPW = E // NW
CHUNK = 80
NCHUNK = EPW // CHUNK
ROWS_PER_TILE = 624
NSTG = 25

BN = 1000
EBN = 80


def _xall_body(wc_ref, w_ref, x_ref, es_ref, et_ref, out_ref, gidx_ref):
  @pl.when(pl.program_id(0) == 0)
  def _():
    gidx_ref[...] = et_ref[...] * N + es_ref[...]
  xb = x_ref[...]
  for r in range(R):
    wr = wc_ref[r, 0] * w_ref[0]
    for b in range(1, B):
      wr = wr + wc_ref[r, b] * w_ref[b]
    out_ref[r] = jnp.dot(xb, wr, preferred_element_type=jnp.float32)


def _combine_body(ws_ref, x_ref, p_ref, out_ref, rep_ref):
  h = (jnp.dot(x_ref[...], ws_ref[...], preferred_element_type=jnp.float32)
       + p_ref[0] + p_ref[1])
  out_ref[...] = h
  rep_ref[:, 0, :] = h


def _sc_body(xall, gidxh, dsth, normh, out,
             gidx_v, norm_v, didx0, didx1, didx2, rows0, rows1, rows2,
             gsem0, gsem1, gsem2, ssem0, ssem1, ssem2,
             dsem0, dsem1, dsem2, isem, acc):
  cid = lax.axis_index("c")
  sid = lax.axis_index("s")
  wid = sid * NC + cid
  base = wid * EPW
  rows = (rows0, rows1, rows2)
  didx = (didx0, didx1, didx2)
  gsem = (gsem0, gsem1, gsem2)
  ssem = (ssem0, ssem1, ssem2)
  dsem = (dsem0, dsem1, dsem2)

  def start_didx(c, b):
    pltpu.async_copy(dsth.at[pl.ds(base + c * CHUNK, CHUNK)], didx[b], dsem[b])

  def wait_didx(b):
    pltpu.make_async_copy(dsth.at[pl.ds(base, CHUNK)], didx[b], dsem[b]).wait()

  def start_gather(c, b):
    pltpu.async_copy(xall.at[gidx_v.at[pl.ds(c * CHUNK, CHUNK)]],
                     rows[b], gsem[b])

  def wait_gather(b):
    pltpu.make_async_copy(xall.at[gidx_v.at[pl.ds(0, CHUNK)]],
                          rows[b], gsem[b]).wait()

  def start_scatter(b):
    pltpu.async_copy(rows[b], acc.at[didx[b]], ssem[b], add=True)

  def wait_scatter(b):
    pltpu.make_async_copy(rows[b], acc.at[didx[b]], ssem[b]).wait()

  pltpu.async_copy(gidxh.at[pl.ds(base, EPW)], gidx_v, isem)

  def zero_body(i, _):
    for k in range(8):
      rows2[i, pl.ds(k * 16, 16)] = jnp.zeros((16,), jnp.float32)
    return 0
  lax.fori_loop(0, CHUNK, zero_body, 0)
  pltpu.make_async_copy(gidxh.at[pl.ds(base, EPW)], gidx_v, isem).wait()
  start_didx(0, 0)
  start_gather(0, 0)
  start_didx(1, 1)
  start_gather(1, 1)
  rbase = pl.multiple_of(sid * ROWS_PER_TILE, 16)
  tail_base = NS * ROWS_PER_TILE
  tail = N - tail_base
  nfull = ROWS_PER_TILE // CHUNK
  rem = ROWS_PER_TILE - nfull * CHUNK
  zslices = [(pl.multiple_of(rbase + k * CHUNK, 8), CHUNK)
             for k in range(nfull)]
  zslices.append((pl.multiple_of(rbase + nfull * CHUNK, 8), rem))
  for off, n in zslices:
    pltpu.async_copy(rows2.at[pl.ds(0, n)], acc.at[pl.ds(off, n)], isem)

  @pl.when(sid == NS - 1)
  def _():
    pltpu.async_copy(rows2.at[pl.ds(0, tail)], acc.at[pl.ds(tail_base, tail)],
                     isem)
  for off, n in zslices:
    pltpu.make_async_copy(rows2.at[pl.ds(0, n)], acc.at[pl.ds(off, n)],
                          isem).wait()

  @pl.when(sid == NS - 1)
  def _():
    pltpu.make_async_copy(rows2.at[pl.ds(0, tail)],
                          acc.at[pl.ds(tail_base, tail)], isem).wait()
  plsc.subcore_barrier()

  def chunk_step(c, par):
    nxt = (par + 2) % 3

    @pl.when(lax.rem(c, NSTG) == 0)
    def _():
      pltpu.sync_copy(
          normh.at[pl.ds(base + lax.div(c, NSTG) * (NSTG * CHUNK),
                         NSTG * CHUNK)],
          norm_v)

    @pl.when(c + 2 < NCHUNK)
    def _():
      @pl.when(c > 0)
      def _():
        wait_scatter(nxt)
      start_didx(c + 2, nxt)
      start_gather(c + 2, nxt)
    wait_gather(par)
    cbase = pl.multiple_of(lax.rem(c, NSTG) * CHUNK, 16)

    def scale_body(g, _):
      nv = norm_v[pl.ds(cbase + g * 16, 16)]
      for t in range(16):
        e = g * 16 + t
        for k in range(8):
          rows[par][e, pl.ds(k * 16, 16)] = (
              rows[par][e, pl.ds(k * 16, 16)] * nv[t])
      return 0
    lax.fori_loop(0, CHUNK // 16, scale_body, 0)
    wait_didx(par)
    start_scatter(par)

  def trip_body(q, _):
    c0 = q * 3
    chunk_step(c0, 0)
    chunk_step(c0 + 1, 1)
    chunk_step(c0 + 2, 2)
    return 0
  lax.fori_loop(0, NCHUNK // 3, trip_body, 0)
  chunk_step(NCHUNK - 2, 0)
  chunk_step(NCHUNK - 1, 1)
  wait_scatter(2)
  wait_scatter(0)
  wait_scatter(1)
  plsc.subcore_barrier()

  for off, n in zslices:
    pltpu.async_copy(acc.at[pl.ds(off, n)], out.at[cid, pl.ds(off, n)], isem)

  @pl.when(sid == NS - 1)
  def _():
    pltpu.async_copy(acc.at[pl.ds(tail_base, tail)],
                     out.at[cid, pl.ds(tail_base, tail)], isem)
  for off, n in zslices:
    pltpu.make_async_copy(acc.at[pl.ds(off, n)],
                          out.at[cid, pl.ds(off, n)], isem).wait()

  @pl.when(sid == NS - 1)
  def _():
    pltpu.make_async_copy(acc.at[pl.ds(tail_base, tail)],
                          out.at[cid, pl.ds(tail_base, tail)], isem).wait()


def kernel(x, norm, rel_emb, weight, w_comp, self_loop_weight,
           edge_src, edge_dst, edge_type, edge_label):
  del rel_emb, edge_label

  grid = N // BN
  x_all, gidx = pl.pallas_call(
      _xall_body,
      grid=(grid,),
      in_specs=[
          pl.BlockSpec(memory_space=pltpu.SMEM),
          pl.BlockSpec((B, D, D), lambda i: (0, 0, 0)),
          pl.BlockSpec((BN, D), lambda i: (i, 0)),
          pl.BlockSpec((E,), lambda i: (0,)),
          pl.BlockSpec((E,), lambda i: (0,)),
      ],
      out_specs=[
          pl.BlockSpec((R, BN, D), lambda i: (0, i, 0)),
          pl.BlockSpec((E,), lambda i: (0,)),
      ],
      out_shape=[
          jax.ShapeDtypeStruct((R, N, D), jnp.float32),
          jax.ShapeDtypeStruct((E,), jnp.int32),
      ],
      compiler_params=pltpu.CompilerParams(
          dimension_semantics=("arbitrary",)),
  )(w_comp, weight, x, edge_src, edge_type)
  xall_flat = x_all.reshape(R * N, D)

  sc_kernel = functools.partial(
      pl.kernel,
      out_type=jax.ShapeDtypeStruct((NC, N, D), jnp.float32),
      mesh=plsc.VectorSubcoreMesh(core_axis_name="c", subcore_axis_name="s"),
      scratch_types=(
          [pltpu.VMEM((EPW,), jnp.int32)]
          + [pltpu.VMEM((NSTG * CHUNK,), jnp.float32)]
          + [pltpu.VMEM((CHUNK,), jnp.int32) for _ in range(3)]
          + [pltpu.VMEM((CHUNK, D), jnp.float32) for _ in range(3)]
          + [pltpu.SemaphoreType.DMA for _ in range(10)]
          + [pltpu.VMEM_SHARED((N, D), jnp.float32)]
      ),
  )(_sc_body)
  partials = sc_kernel(xall_flat, gidx, edge_dst, norm)

  h, rep = pl.pallas_call(
      _combine_body,
      grid=(grid,),
      in_specs=[
          pl.BlockSpec((D, D), lambda i: (0, 0)),
          pl.BlockSpec((BN, D), lambda i: (i, 0)),
          pl.BlockSpec((NC, BN, D), lambda i: (0, i, 0)),
      ],
      out_specs=[
          pl.BlockSpec((BN, D), lambda i: (i, 0)),
          pl.BlockSpec((BN, 1, D), lambda i: (i, 0, 0)),
      ],
      out_shape=[
          jax.ShapeDtypeStruct((N, D), jnp.float32),
          jax.ShapeDtypeStruct((N, 1, D), jnp.float32),
      ],
      compiler_params=pltpu.CompilerParams(
          dimension_semantics=("arbitrary",)),
  )(self_loop_weight, x, partials)

  return h, rep


# speedup vs baseline: 1.3297x; 1.0006x over previous
import functools

import jax
import jax.numpy as jnp
from jax import lax
from jax.experimental import pallas as pl
from jax.experimental.pallas import tpu as pltpu
from jax.experimental.pallas import tpu_sc as plsc

N = 10000
E = 320000
D = 128
R = 8
B = 4

NC = 2
NS = 16
NW = NC * NS
EPW = E // NW
CHUNK = 80
NCHUNK = EPW // CHUNK
ROWS_PER_TILE = 624
NSTG = 25

BN = 1000
EBN = 80


def _xall_body(wc_ref, w_ref, x_ref, es_ref, et_ref, out_ref, gidx_ref):
  @pl.when(pl.program_id(0) == 0)
  def _():
    gidx_ref[...] = et_ref[...] * N + es_ref[...]
  xb = x_ref[...]
  for r in range(R):
    wr = wc_ref[r, 0] * w_ref[0]
    for b in range(1, B):
      wr = wr + wc_ref[r, b] * w_ref[b]
    out_ref[r] = jnp.dot(xb, wr, preferred_element_type=jnp.float32)


def _combine_body(ws_ref, x_ref, p_ref, out_ref, rep_ref):
  h = (jnp.dot(x_ref[...], ws_ref[...], preferred_element_type=jnp.float32)
       + p_ref[0] + p_ref[1])
  out_ref[...] = h
  rep_ref[:, 0, :] = h


def _sc_body(xall, gidxh, dsth, normh, out,
             gidx_v, norm_v, dstb0, dstb1, rows0, rows1, rows2,
             gsem0, gsem1, gsem2, ssem0, ssem1, ssem2,
             dsem0, dsem1, isem, acc):
  cid = lax.axis_index("c")
  sid = lax.axis_index("s")
  wid = sid * NC + cid
  base = wid * EPW
  rows = (rows0, rows1, rows2)
  dstb = (dstb0, dstb1)
  gsem = (gsem0, gsem1, gsem2)
  ssem = (ssem0, ssem1, ssem2)
  dsem = (dsem0, dsem1)

  def start_dst_stage(s, b):
    pltpu.async_copy(dsth.at[pl.ds(base + s * (NSTG * CHUNK), NSTG * CHUNK)],
                     dstb[b], dsem[b])

  def wait_dst_stage(b):
    pltpu.make_async_copy(dsth.at[pl.ds(base, NSTG * CHUNK)], dstb[b],
                          dsem[b]).wait()

  def start_gather(c, b):
    pltpu.async_copy(xall.at[gidx_v.at[pl.ds(c * CHUNK, CHUNK)]],
                     rows[b], gsem[b])

  def wait_gather(b):
    pltpu.make_async_copy(xall.at[gidx_v.at[pl.ds(0, CHUNK)]],
                          rows[b], gsem[b]).wait()

  def dst_slice(c):
    sp = lax.rem(lax.div(c, NSTG), 2)
    lp = pl.multiple_of(lax.rem(c, NSTG) * CHUNK, 16)
    return sp, lp

  def start_scatter(c, b):
    sp, lp = dst_slice(c)
    pltpu.async_copy(rows[b], acc.at[dstb[0].at[pl.ds(lp, CHUNK)]], ssem[b],
                     add=True) if False else None
    @pl.when(sp == 0)
    def _():
      pltpu.async_copy(rows[b], acc.at[dstb[0].at[pl.ds(lp, CHUNK)]],
                       ssem[b], add=True)
    @pl.when(sp == 1)
    def _():
      pltpu.async_copy(rows[b], acc.at[dstb[1].at[pl.ds(lp, CHUNK)]],
                       ssem[b], add=True)

  def wait_scatter_for(c, b):
    sp, lp = dst_slice(c)
    @pl.when(sp == 0)
    def _():
      pltpu.make_async_copy(rows[b], acc.at[dstb[0].at[pl.ds(lp, CHUNK)]],
                            ssem[b]).wait()
    @pl.when(sp == 1)
    def _():
      pltpu.make_async_copy(rows[b], acc.at[dstb[1].at[pl.ds(lp, CHUNK)]],
                            ssem[b]).wait()

  pltpu.async_copy(gidxh.at[pl.ds(base, EPW)], gidx_v, isem)
  start_dst_stage(0, 0)

  def zero_body(i, _):
    for k in range(8):
      rows2[i, pl.ds(k * 16, 16)] = jnp.zeros((16,), jnp.float32)
    return 0
  lax.fori_loop(0, CHUNK, zero_body, 0)
  pltpu.make_async_copy(gidxh.at[pl.ds(base, EPW)], gidx_v, isem).wait()
  start_gather(0, 0)
  start_gather(1, 1)
  wait_dst_stage(0)
  rbase = pl.multiple_of(sid * ROWS_PER_TILE, 16)
  tail_base = NS * ROWS_PER_TILE
  tail = N - tail_base
  nfull = ROWS_PER_TILE // CHUNK
  rem = ROWS_PER_TILE - nfull * CHUNK
  zslices = [(pl.multiple_of(rbase + k * CHUNK, 8), CHUNK)
             for k in range(nfull)]
  zslices.append((pl.multiple_of(rbase + nfull * CHUNK, 8), rem))
  for off, n in zslices:
    pltpu.async_copy(rows2.at[pl.ds(0, n)], acc.at[pl.ds(off, n)], isem)

  @pl.when(sid == NS - 1)
  def _():
    pltpu.async_copy(rows2.at[pl.ds(0, tail)], acc.at[pl.ds(tail_base, tail)],
                     isem)
  for off, n in zslices:
    pltpu.make_async_copy(rows2.at[pl.ds(0, n)], acc.at[pl.ds(off, n)],
                          isem).wait()

  @pl.when(sid == NS - 1)
  def _():
    pltpu.make_async_copy(rows2.at[pl.ds(0, tail)],
                          acc.at[pl.ds(tail_base, tail)], isem).wait()
  plsc.subcore_barrier()

  def chunk_step(c, par):
    nxt = (par + 2) % 3

    @pl.when(lax.rem(c, NSTG) == 0)
    def _():
      pltpu.sync_copy(
          normh.at[pl.ds(base + lax.div(c, NSTG) * (NSTG * CHUNK),
                         NSTG * CHUNK)],
          norm_v)

    sparity = lax.rem(lax.div(c, NSTG), 2)

    @pl.when((lax.rem(c, NSTG) == 0) & (c > 0) & (sparity == 0))
    def _():
      wait_dst_stage(0)

    @pl.when((lax.rem(c, NSTG) == 0) & (c > 0) & (sparity == 1))
    def _():
      wait_dst_stage(1)

    @pl.when((lax.rem(c, NSTG) == NSTG - 2) & (c < NCHUNK - NSTG)
             & (sparity == 1))
    def _():
      start_dst_stage(lax.div(c, NSTG) + 1, 0)

    @pl.when((lax.rem(c, NSTG) == NSTG - 2) & (c < NCHUNK - NSTG)
             & (sparity == 0))
    def _():
      start_dst_stage(lax.div(c, NSTG) + 1, 1)

    @pl.when(c + 2 < NCHUNK)
    def _():
      @pl.when(c > 0)
      def _():
        wait_scatter_for(c - 1, nxt)
      start_gather(c + 2, nxt)
    wait_gather(par)
    cbase = pl.multiple_of(lax.rem(c, NSTG) * CHUNK, 16)

    def scale_body(g, _):
      nv = norm_v[pl.ds(cbase + g * 16, 16)]
      for t in range(16):
        e = g * 16 + t
        for k in range(8):
          rows[par][e, pl.ds(k * 16, 16)] = (
              rows[par][e, pl.ds(k * 16, 16)] * nv[t])
      return 0
    lax.fori_loop(0, CHUNK // 16, scale_body, 0)
    start_scatter(c, par)

  def trip_body(q, _):
    c0 = q * 3
    chunk_step(c0, 0)
    chunk_step(c0 + 1, 1)
    chunk_step(c0 + 2, 2)
    return 0
  lax.fori_loop(0, NCHUNK // 3, trip_body, 0)
  chunk_step(NCHUNK - 2, 0)
  chunk_step(NCHUNK - 1, 1)
  wait_scatter_for(NCHUNK - 3, 2)
  wait_scatter_for(NCHUNK - 2, 0)
  wait_scatter_for(NCHUNK - 1, 1)
  plsc.subcore_barrier()

  for off, n in zslices:
    pltpu.async_copy(acc.at[pl.ds(off, n)], out.at[cid, pl.ds(off, n)], isem)

  @pl.when(sid == NS - 1)
  def _():
    pltpu.async_copy(acc.at[pl.ds(tail_base, tail)],
                     out.at[cid, pl.ds(tail_base, tail)], isem)
  for off, n in zslices:
    pltpu.make_async_copy(acc.at[pl.ds(off, n)],
                          out.at[cid, pl.ds(off, n)], isem).wait()

  @pl.when(sid == NS - 1)
  def _():
    pltpu.make_async_copy(acc.at[pl.ds(tail_base, tail)],
                          out.at[cid, pl.ds(tail_base, tail)], isem).wait()


def kernel(x, norm, rel_emb, weight, w_comp, self_loop_weight,
           edge_src, edge_dst, edge_type, edge_label):
  del rel_emb, edge_label

  grid = N // BN
  x_all, gidx = pl.pallas_call(
      _xall_body,
      grid=(grid,),
      in_specs=[
          pl.BlockSpec(memory_space=pltpu.SMEM),
          pl.BlockSpec((B, D, D), lambda i: (0, 0, 0)),
          pl.BlockSpec((BN, D), lambda i: (i, 0)),
          pl.BlockSpec((E,), lambda i: (0,)),
          pl.BlockSpec((E,), lambda i: (0,)),
      ],
      out_specs=[
          pl.BlockSpec((R, BN, D), lambda i: (0, i, 0)),
          pl.BlockSpec((E,), lambda i: (0,)),
      ],
      out_shape=[
          jax.ShapeDtypeStruct((R, N, D), jnp.float32),
          jax.ShapeDtypeStruct((E,), jnp.int32),
      ],
      compiler_params=pltpu.CompilerParams(
          dimension_semantics=("arbitrary",)),
  )(w_comp, weight, x, edge_src, edge_type)
  xall_flat = x_all.reshape(R * N, D)

  sc_kernel = functools.partial(
      pl.kernel,
      out_type=jax.ShapeDtypeStruct((NC, N, D), jnp.float32),
      mesh=plsc.VectorSubcoreMesh(core_axis_name="c", subcore_axis_name="s"),
      scratch_types=(
          [pltpu.VMEM((EPW,), jnp.int32)]
          + [pltpu.VMEM((NSTG * CHUNK,), jnp.float32)]
          + [pltpu.VMEM((NSTG * CHUNK,), jnp.int32) for _ in range(2)]
          + [pltpu.VMEM((CHUNK, D), jnp.float32) for _ in range(3)]
          + [pltpu.SemaphoreType.DMA for _ in range(9)]
          + [pltpu.VMEM_SHARED((N, D), jnp.float32)]
      ),
  )(_sc_body)
  partials = sc_kernel(xall_flat, gidx, edge_dst, norm)

  h, rep = pl.pallas_call(
      _combine_body,
      grid=(grid,),
      in_specs=[
          pl.BlockSpec((D, D), lambda i: (0, 0)),
          pl.BlockSpec((BN, D), lambda i: (i, 0)),
          pl.BlockSpec((NC, BN, D), lambda i: (0, i, 0)),
      ],
      out_specs=[
          pl.BlockSpec((BN, D), lambda i: (i, 0)),
          pl.BlockSpec((BN, 1, D), lambda i: (i, 0, 0)),
      ],
      out_shape=[
          jax.ShapeDtypeStruct((N, D), jnp.float32),
          jax.ShapeDtypeStruct((N, 1, D), jnp.float32),
      ],
      compiler_params=pltpu.CompilerParams(
          dimension_semantics=("arbitrary",)),
  )(self_loop_weight, x, partials)

  return h, rep
